# baseline (speedup 1.0000x reference)
import functools

import numpy as np
import jax
import jax.numpy as jnp
from jax import lax
from jax.experimental import pallas as pl
from jax.experimental.pallas import tpu as pltpu

_B, _C, _T, _H, _W = 4, 3, 32, 224, 224
_K = 8
_R2, _R3 = (_H * _W) // 128, 128


def _sorted_inds() -> np.ndarray:
    idx_top = np.linspace(0, _T, _K + 1).astype(np.int32)[:-1]
    idx_back = np.array(sorted(set(range(_T)) - set(idx_top.tolist())),
                        dtype=np.int32)
    return np.tile(np.concatenate([idx_top, idx_back])[None, :], (_B, 1))


_SORTED_INDS = _sorted_inds()


def _copy_body(x_ref, o_ref):
    o_ref[...] = x_ref[...]


@jax.jit
def _tc_permute(x3d):
    top = pl.pallas_call(
        _copy_body,
        grid=(96,),
        in_specs=[pl.BlockSpec(
            (1, _R2, _R3),
            lambda i: (32 * (i // _K) + 4 * (i % _K), 0, 0))],
        out_specs=pl.BlockSpec((1, _R2, _R3), lambda i: (i, 0, 0)),
        out_shape=jax.ShapeDtypeStruct((96, _R2, _R3), jnp.float32),
    )(x3d)
    back = pl.pallas_call(
        _copy_body,
        grid=(288,),
        in_specs=[pl.BlockSpec(
            (1, _R2, _R3),
            lambda j: (32 * (j // 24) + 4 * ((j % 24) // 3) + (j % 24) % 3 + 1,
                       0, 0))],
        out_specs=pl.BlockSpec((1, _R2, _R3), lambda j: (j, 0, 0)),
        out_shape=jax.ShapeDtypeStruct((288, _R2, _R3), jnp.float32),
    )(x3d)
    return top, back


def kernel(frames):
    x3d = frames.reshape(_B * _C * _T, _R2, _R3)
    top, back = _tc_permute(x3d)
    frames_topk = top.reshape(_B, _C, _K, _H, _W)
    frames_back = back.reshape(_B, _C, _T - _K, _H, _W)
    return frames_topk, frames_back, jnp.asarray(_SORTED_INDS)
